# baseline (speedup 1.0000x reference)
import functools

import jax
import jax.numpy as jnp
from jax import lax
from jax.experimental import pallas as pl
from jax.experimental.pallas import tpu as pltpu


def _fold_bn(gamma, beta, mean, var, eps=1e-5):
    scale = gamma / jnp.sqrt(var + eps)
    return scale, beta - mean * scale


def _wprep_kernel(w1_ref, w2_ref, w1k_ref, w2k_ref, eye_ref, *, C):
    K = 9 * C
    ri = lax.broadcasted_iota(jnp.int32, (C, C), 0)
    ci = lax.broadcasted_iota(jnp.int32, (C, C), 1)
    eye = (ri == ci).astype(jnp.bfloat16)
    eye_ref[...] = eye
    rp = lax.broadcasted_iota(jnp.int32, (K, K), 0)
    cp = lax.broadcasted_iota(jnp.int32, (K, K), 1)
    perm = (cp == (rp % C) * 9 + rp // C).astype(jnp.bfloat16)
    for wref, okref in ((w1_ref, w1k_ref), (w2_ref, w2k_ref)):
        wb = wref[...].astype(jnp.bfloat16)
        wt = lax.dot_general(wb, eye, (((0,), (0,)), ((), ())),
                             preferred_element_type=jnp.float32)
        wk = lax.dot_general(perm, wt.astype(jnp.bfloat16),
                             (((1,), (0,)), ((), ())),
                             preferred_element_type=jnp.float32)
        okref[...] = wk.astype(jnp.bfloat16)


def _block_kernel(x_ref, w1_ref, w2_ref, bn_ref, eye_ref, o_ref, *, H, W, C):
    HW = H * W
    x = x_ref[...]

    s1 = bn_ref[:, 0:1]
    b1 = bn_ref[:, 1:2]
    s2 = bn_ref[:, 2:3]
    b2 = bn_ref[:, 3:4]
    eye = eye_ref[...]

    def taps(y_cm):
        yb = y_cm.astype(jnp.bfloat16)
        yt = lax.dot_general(yb, eye, (((0,), (0,)), ((), ())),
                             preferred_element_type=jnp.float32)
        g = jnp.pad(yt.astype(jnp.bfloat16).reshape(H, W, C),
                    ((1, 1), (1, 1), (0, 0)))
        return jnp.concatenate(
            [g[dy:dy + H, dx:dx + W, :].reshape(HW, C)
             for dy in range(3) for dx in range(3)], axis=1)

    cols1 = taps(jnp.maximum(x * s1 + b1, 0.0))
    acc1 = lax.dot_general(w1_ref[...], cols1, (((0,), (1,)), ((), ())),
                           preferred_element_type=jnp.float32)

    cols2 = taps(jnp.maximum(acc1 * s2 + b2, 0.0))
    acc2 = lax.dot_general(w2_ref[...], cols2, (((0,), (1,)), ((), ())),
                           preferred_element_type=jnp.float32)

    o_ref[...] = x + acc2


@jax.jit
def _basic_unit(x_nchw, w1, w2, bn1, bn2):
    n, c, h, w = x_nchw.shape
    hw = h * w
    x2d = x_nchw.reshape(n, c, hw)

    s1, b1 = _fold_bn(*bn1)
    s2, b2 = _fold_bn(*bn2)
    bn = jnp.stack([s1, b1, s2, b2], axis=1)
    bn = jnp.pad(bn, ((0, 0), (0, 4)))

    w1k, w2k, eye = pl.pallas_call(
        functools.partial(_wprep_kernel, C=c),
        out_shape=[
            jax.ShapeDtypeStruct((9 * c, c), jnp.bfloat16),
            jax.ShapeDtypeStruct((9 * c, c), jnp.bfloat16),
            jax.ShapeDtypeStruct((c, c), jnp.bfloat16),
        ],
    )(w1.reshape(c, 9 * c), w2.reshape(c, 9 * c))

    kfn = functools.partial(_block_kernel, H=h, W=w, C=c)
    out2d = pl.pallas_call(
        kfn,
        out_shape=jax.ShapeDtypeStruct((n, c, hw), jnp.float32),
        grid=(n,),
        in_specs=[
            pl.BlockSpec((None, c, hw), lambda i: (i, 0, 0)),
            pl.BlockSpec((9 * c, c), lambda i: (0, 0)),
            pl.BlockSpec((9 * c, c), lambda i: (0, 0)),
            pl.BlockSpec((c, 8), lambda i: (0, 0)),
            pl.BlockSpec((c, c), lambda i: (0, 0)),
        ],
        out_specs=pl.BlockSpec((None, c, hw), lambda i: (i, 0, 0)),
        compiler_params=pltpu.CompilerParams(
            dimension_semantics=("parallel",),
            vmem_limit_bytes=64 * 1024 * 1024,
        ),
    )(x2d, w1k, w2k, bn, eye)

    return out2d.reshape(n, c, h, w)


def kernel(x, w1, w2, bn1_gamma, bn1_beta, bn1_mean, bn1_var,
           bn2_gamma, bn2_beta, bn2_mean, bn2_var):
    return _basic_unit(x, w1, w2,
                       (bn1_gamma, bn1_beta, bn1_mean, bn1_var),
                       (bn2_gamma, bn2_beta, bn2_mean, bn2_var))


# speedup vs baseline: 1.2965x; 1.2965x over previous
import functools

import jax
import jax.numpy as jnp
from jax import lax
from jax.experimental import pallas as pl
from jax.experimental.pallas import tpu as pltpu


def _fold_bn(gamma, beta, mean, var, eps=1e-5):
    scale = gamma / jnp.sqrt(var + eps)
    return scale, beta - mean * scale


def _block_kernel(x_ref, w1_ref, w2_ref, bn_ref, o_ref, *, H, W, C):
    HW = H * W
    x = x_ref[...]

    s1 = bn_ref[0:1, :]
    b1 = bn_ref[1:2, :]
    s2 = bn_ref[2:3, :]
    b2 = bn_ref[3:4, :]

    def cols_of(y):
        g = jnp.pad(y.astype(jnp.bfloat16).reshape(H, W, C),
                    ((1, 1), (1, 1), (0, 0)))
        return jnp.concatenate(
            [g[dy:dy + H, dx:dx + W, :].reshape(HW, C)
             for dy in range(3) for dx in range(3)], axis=1)

    cols1 = cols_of(jnp.maximum(x * s1 + b1, 0.0))
    acc1 = jnp.dot(cols1, w1_ref[...], preferred_element_type=jnp.float32)

    cols2 = cols_of(jnp.maximum(acc1 * s2 + b2, 0.0))
    acc2 = jnp.dot(cols2, w2_ref[...], preferred_element_type=jnp.float32)

    o_ref[...] = x + acc2


@jax.jit
def _basic_unit(x_nchw, w1, w2, bn1, bn2):
    n, c, h, w = x_nchw.shape
    hw = h * w
    x2d = jnp.transpose(x_nchw, (0, 2, 3, 1)).reshape(n, hw, c)

    s1, b1 = _fold_bn(*bn1)
    s2, b2 = _fold_bn(*bn2)
    bn = jnp.zeros((8, c), jnp.float32)
    bn = bn.at[0].set(s1).at[1].set(b1).at[2].set(s2).at[3].set(b2)

    def prep_w(wt):
        wk = jnp.transpose(wt, (2, 3, 1, 0)).reshape(9 * c, c)
        return wk.astype(jnp.bfloat16)

    w1k = prep_w(w1)
    w2k = prep_w(w2)

    kfn = functools.partial(_block_kernel, H=h, W=w, C=c)
    out2d = pl.pallas_call(
        kfn,
        out_shape=jax.ShapeDtypeStruct((n, hw, c), jnp.float32),
        grid=(n,),
        in_specs=[
            pl.BlockSpec((None, hw, c), lambda i: (i, 0, 0)),
            pl.BlockSpec((9 * c, c), lambda i: (0, 0)),
            pl.BlockSpec((9 * c, c), lambda i: (0, 0)),
            pl.BlockSpec((8, c), lambda i: (0, 0)),
        ],
        out_specs=pl.BlockSpec((None, hw, c), lambda i: (i, 0, 0)),
        compiler_params=pltpu.CompilerParams(
            dimension_semantics=("parallel",),
            vmem_limit_bytes=64 * 1024 * 1024,
        ),
    )(x2d, w1k, w2k, bn)

    out = out2d.reshape(n, h, w, c)
    return jnp.transpose(out, (0, 3, 1, 2))


def kernel(x, w1, w2, bn1_gamma, bn1_beta, bn1_mean, bn1_var,
           bn2_gamma, bn2_beta, bn2_mean, bn2_var):
    return _basic_unit(x, w1, w2,
                       (bn1_gamma, bn1_beta, bn1_mean, bn1_var),
                       (bn2_gamma, bn2_beta, bn2_mean, bn2_var))


# speedup vs baseline: 1.3910x; 1.0729x over previous
import functools

import jax
import jax.numpy as jnp
from jax import lax
from jax.experimental import pallas as pl
from jax.experimental.pallas import tpu as pltpu


def _fold_bn(gamma, beta, mean, var, eps=1e-5):
    scale = gamma / jnp.sqrt(var + eps)
    return scale, beta - mean * scale


def _block_kernel(x_ref, w1_ref, w2_ref, bn_ref, o_ref, *, H, W, C, B):
    HW = H * W
    x = x_ref[...].reshape(B * HW, C)

    s1 = bn_ref[0:1, :]
    b1 = bn_ref[1:2, :]
    s2 = bn_ref[2:3, :]
    b2 = bn_ref[3:4, :]

    def cols_of(y):
        g = jnp.pad(y.astype(jnp.bfloat16).reshape(B, H, W, C),
                    ((0, 0), (1, 1), (1, 1), (0, 0)))
        return jnp.concatenate(
            [g[:, dy:dy + H, dx:dx + W, :].reshape(B * HW, C)
             for dy in range(3) for dx in range(3)], axis=1)

    cols1 = cols_of(jnp.maximum(x * s1 + b1, 0.0))
    acc1 = jnp.dot(cols1, w1_ref[...], preferred_element_type=jnp.float32)

    cols2 = cols_of(jnp.maximum(acc1 * s2 + b2, 0.0))
    acc2 = jnp.dot(cols2, w2_ref[...], preferred_element_type=jnp.float32)

    o_ref[...] = (x + acc2).reshape(B, HW, C)


@jax.jit
def _basic_unit(x_nchw, w1, w2, bn1, bn2):
    n, c, h, w = x_nchw.shape
    hw = h * w
    x2d = jnp.transpose(x_nchw, (0, 2, 3, 1)).reshape(n, hw, c)

    s1, b1 = _fold_bn(*bn1)
    s2, b2 = _fold_bn(*bn2)
    bn = jnp.zeros((8, c), jnp.float32)
    bn = bn.at[0].set(s1).at[1].set(b1).at[2].set(s2).at[3].set(b2)

    def prep_w(wt):
        wk = jnp.transpose(wt, (2, 3, 1, 0)).reshape(9 * c, c)
        return wk.astype(jnp.bfloat16)

    w1k = prep_w(w1)
    w2k = prep_w(w2)

    b = 2
    kfn = functools.partial(_block_kernel, H=h, W=w, C=c, B=b)
    out2d = pl.pallas_call(
        kfn,
        out_shape=jax.ShapeDtypeStruct((n, hw, c), jnp.float32),
        grid=(n // b,),
        in_specs=[
            pl.BlockSpec((b, hw, c), lambda i: (i, 0, 0)),
            pl.BlockSpec((9 * c, c), lambda i: (0, 0)),
            pl.BlockSpec((9 * c, c), lambda i: (0, 0)),
            pl.BlockSpec((8, c), lambda i: (0, 0)),
        ],
        out_specs=pl.BlockSpec((b, hw, c), lambda i: (i, 0, 0)),
        compiler_params=pltpu.CompilerParams(
            dimension_semantics=("parallel",),
            vmem_limit_bytes=64 * 1024 * 1024,
        ),
    )(x2d, w1k, w2k, bn)

    out = out2d.reshape(n, h, w, c)
    return jnp.transpose(out, (0, 3, 1, 2))


def kernel(x, w1, w2, bn1_gamma, bn1_beta, bn1_mean, bn1_var,
           bn2_gamma, bn2_beta, bn2_mean, bn2_var):
    return _basic_unit(x, w1, w2,
                       (bn1_gamma, bn1_beta, bn1_mean, bn1_var),
                       (bn2_gamma, bn2_beta, bn2_mean, bn2_var))


# speedup vs baseline: 1.4426x; 1.0370x over previous
import functools

import jax
import jax.numpy as jnp
from jax import lax
from jax.experimental import pallas as pl
from jax.experimental.pallas import tpu as pltpu


def _fold_bn(gamma, beta, mean, var, eps=1e-5):
    scale = gamma / jnp.sqrt(var + eps)
    return scale, beta - mean * scale


def _block_kernel(x_ref, w1_ref, w2_ref, bn_ref, o_ref, *, H, W, C, B):
    HW = H * W
    x = x_ref[...].reshape(B * HW, C)

    s1 = bn_ref[0:1, :]
    b1 = bn_ref[1:2, :]
    s2 = bn_ref[2:3, :]
    b2 = bn_ref[3:4, :]

    def cols_of(y):
        g = jnp.pad(y.astype(jnp.bfloat16).reshape(B, H, W, C),
                    ((0, 0), (1, 1), (1, 1), (0, 0)))
        return jnp.concatenate(
            [g[:, dy:dy + H, dx:dx + W, :].reshape(B * HW, C)
             for dy in range(3) for dx in range(3)], axis=1)

    cols1 = cols_of(jnp.maximum(x * s1 + b1, 0.0))
    acc1 = jnp.dot(cols1, w1_ref[...], preferred_element_type=jnp.float32)

    cols2 = cols_of(jnp.maximum(acc1 * s2 + b2, 0.0))
    acc2 = jnp.dot(cols2, w2_ref[...], preferred_element_type=jnp.float32)

    o_ref[...] = (x + acc2).reshape(B, HW, C)


@jax.jit
def _basic_unit(x_nchw, w1, w2, bn1, bn2):
    n, c, h, w = x_nchw.shape
    hw = h * w
    x2d = jnp.transpose(x_nchw, (0, 2, 3, 1)).reshape(n, hw, c)

    s1, b1 = _fold_bn(*bn1)
    s2, b2 = _fold_bn(*bn2)
    bn = jnp.zeros((8, c), jnp.float32)
    bn = bn.at[0].set(s1).at[1].set(b1).at[2].set(s2).at[3].set(b2)

    def prep_w(wt):
        wk = jnp.transpose(wt, (2, 3, 1, 0)).reshape(9 * c, c)
        return wk.astype(jnp.bfloat16)

    w1k = prep_w(w1)
    w2k = prep_w(w2)

    b = 4
    kfn = functools.partial(_block_kernel, H=h, W=w, C=c, B=b)
    out2d = pl.pallas_call(
        kfn,
        out_shape=jax.ShapeDtypeStruct((n, hw, c), jnp.float32),
        grid=(n // b,),
        in_specs=[
            pl.BlockSpec((b, hw, c), lambda i: (i, 0, 0)),
            pl.BlockSpec((9 * c, c), lambda i: (0, 0)),
            pl.BlockSpec((9 * c, c), lambda i: (0, 0)),
            pl.BlockSpec((8, c), lambda i: (0, 0)),
        ],
        out_specs=pl.BlockSpec((b, hw, c), lambda i: (i, 0, 0)),
        compiler_params=pltpu.CompilerParams(
            dimension_semantics=("parallel",),
            vmem_limit_bytes=64 * 1024 * 1024,
        ),
    )(x2d, w1k, w2k, bn)

    out = out2d.reshape(n, h, w, c)
    return jnp.transpose(out, (0, 3, 1, 2))


def kernel(x, w1, w2, bn1_gamma, bn1_beta, bn1_mean, bn1_var,
           bn2_gamma, bn2_beta, bn2_mean, bn2_var):
    return _basic_unit(x, w1, w2,
                       (bn1_gamma, bn1_beta, bn1_mean, bn1_var),
                       (bn2_gamma, bn2_beta, bn2_mean, bn2_var))


# speedup vs baseline: 1.4624x; 1.0138x over previous
import functools

import jax
import jax.numpy as jnp
from jax import lax
from jax.experimental import pallas as pl
from jax.experimental.pallas import tpu as pltpu


def _fold_bn(gamma, beta, mean, var, eps=1e-5):
    scale = gamma / jnp.sqrt(var + eps)
    return scale, beta - mean * scale


def _block_kernel(x_ref, w1_ref, w2_ref, bn_ref, o_ref, *, H, W, C, B):
    HW = H * W
    x = x_ref[...].reshape(B * HW, C)

    s1 = bn_ref[0:1, :]
    b1 = bn_ref[1:2, :]
    s2 = bn_ref[2:3, :]
    b2 = bn_ref[3:4, :]

    def cols_of(y):
        g = jnp.pad(y.astype(jnp.bfloat16).reshape(B, H, W, C),
                    ((0, 0), (1, 1), (1, 1), (0, 0)))
        return jnp.concatenate(
            [g[:, dy:dy + H, dx:dx + W, :].reshape(B * HW, C)
             for dy in range(3) for dx in range(3)], axis=1)

    cols1 = cols_of(jnp.maximum(x * s1 + b1, 0.0))
    acc1 = jnp.dot(cols1, w1_ref[...], preferred_element_type=jnp.float32)

    cols2 = cols_of(jnp.maximum(acc1 * s2 + b2, 0.0))
    acc2 = jnp.dot(cols2, w2_ref[...], preferred_element_type=jnp.float32)

    o_ref[...] = (x + acc2).reshape(B, HW, C)


@jax.jit
def _basic_unit(x_nchw, w1, w2, bn1, bn2):
    n, c, h, w = x_nchw.shape
    hw = h * w
    x2d = jnp.transpose(x_nchw, (0, 2, 3, 1)).reshape(n, hw, c)

    s1, b1 = _fold_bn(*bn1)
    s2, b2 = _fold_bn(*bn2)
    bn = jnp.zeros((8, c), jnp.float32)
    bn = bn.at[0].set(s1).at[1].set(b1).at[2].set(s2).at[3].set(b2)

    def prep_w(wt):
        wk = jnp.transpose(wt, (2, 3, 1, 0)).reshape(9 * c, c)
        return wk.astype(jnp.bfloat16)

    w1k = prep_w(w1)
    w2k = prep_w(w2)

    b = 8
    kfn = functools.partial(_block_kernel, H=h, W=w, C=c, B=b)
    out2d = pl.pallas_call(
        kfn,
        out_shape=jax.ShapeDtypeStruct((n, hw, c), jnp.float32),
        grid=(n // b,),
        in_specs=[
            pl.BlockSpec((b, hw, c), lambda i: (i, 0, 0)),
            pl.BlockSpec((9 * c, c), lambda i: (0, 0)),
            pl.BlockSpec((9 * c, c), lambda i: (0, 0)),
            pl.BlockSpec((8, c), lambda i: (0, 0)),
        ],
        out_specs=pl.BlockSpec((b, hw, c), lambda i: (i, 0, 0)),
        compiler_params=pltpu.CompilerParams(
            dimension_semantics=("parallel",),
            vmem_limit_bytes=64 * 1024 * 1024,
        ),
    )(x2d, w1k, w2k, bn)

    out = out2d.reshape(n, h, w, c)
    return jnp.transpose(out, (0, 3, 1, 2))


def kernel(x, w1, w2, bn1_gamma, bn1_beta, bn1_mean, bn1_var,
           bn2_gamma, bn2_beta, bn2_mean, bn2_var):
    return _basic_unit(x, w1, w2,
                       (bn1_gamma, bn1_beta, bn1_mean, bn1_var),
                       (bn2_gamma, bn2_beta, bn2_mean, bn2_var))
